# Initial kernel scaffold; baseline (speedup 1.0000x reference)
import jax
import jax.numpy as jnp
from jax.experimental import pallas as pl


def kernel(inputs, edge_index, W0, al0, ar0, b0, W1, al1, ar1, b1, W2, al2, ar2, b2, Wres2):
    raise NotImplementedError("write your pallas kernel here")


# speedup vs baseline: 15.7942x; 15.7942x over previous
import functools

import jax
import jax.numpy as jnp
from jax import lax
from jax.experimental import pallas as pl
from jax.experimental.pallas import tpu as pltpu
from jax.experimental.pallas import tpu_sc as plsc

_N = 10000
_E = 320000
_H = 128
_C = 64
_NP = 10240
_EP = 323584
_EPT = _EP // 32
_BE = 128
_NBLK = _EPT // _BE
_ROWS_PT = _NP // 16
_NZC = _ROWS_PT // _BE
_NEG = 0.2


@functools.cache
def _make_sc_layer(D):
    mesh = plsc.VectorSubcoreMesh(core_axis_name="c", subcore_axis_name="s")

    @functools.partial(
        pl.kernel,
        out_type=[
            jax.ShapeDtypeStruct((2, _NP, D), jnp.float32),
            jax.ShapeDtypeStruct((2, _NP), jnp.float32),
            jax.ShapeDtypeStruct((2, 16), jnp.float32),
        ],
        mesh=mesh,
        scratch_types=[
            pltpu.VMEM((_N,), jnp.float32),
            pltpu.VMEM((_N,), jnp.float32),
            pltpu.VMEM((_BE, D), jnp.float32),
            pltpu.VMEM((_BE,), jnp.int32),
            pltpu.VMEM((_BE,), jnp.int32),
            pltpu.VMEM((_BE,), jnp.float32),
            pltpu.VMEM((_BE,), jnp.float32),
            pltpu.VMEM((16, 16), jnp.float32),
            pltpu.VMEM((16,), jnp.float32),
            pltpu.VMEM_SHARED((_NP, D), jnp.float32),
            pltpu.VMEM_SHARED((_NP,), jnp.float32),
            pltpu.VMEM_SHARED((16, 16), jnp.float32),
            pltpu.SemaphoreType.DMA,
        ],
        compiler_params=pltpu.CompilerParams(needs_layout_passes=False),
    )
    def sc_fn(feat_h, el_h, er_h, src_h, dst_h, p_h, d_h, m_h,
              el_v, er_v, rows_v, srcblk_v, dstblk_v, eeblk_v,
              zrow_v, mall_v, mbuf_v, sh_p, sh_d, sh_m, sem):
        c = lax.axis_index("c")
        s = lax.axis_index("s")
        wid = c * 16 + s
        ebase = wid * _EPT

        pltpu.sync_copy(el_h, el_v)
        pltpu.sync_copy(er_h, er_v)

        iota16 = lax.broadcasted_iota(jnp.int32, (16,), 0)
        zero16 = jnp.zeros((16,), jnp.float32)

        def edge_scores(eb, j):
            sl = pl.ds(j * 16, 16)
            e = (plsc.load_gather(el_v, [srcblk_v[sl]])
                 + plsc.load_gather(er_v, [dstblk_v[sl]]))
            e = jnp.where(e >= 0.0, e, _NEG * e)
            gid = eb + j * 16 + iota16
            return jnp.where(gid < _E, e, -1e30)

        def body_a(b, mcur):
            eb = ebase + b * _BE
            pltpu.sync_copy(src_h.at[pl.ds(eb, _BE)], srcblk_v)
            pltpu.sync_copy(dst_h.at[pl.ds(eb, _BE)], dstblk_v)
            for j in range(_BE // 16):
                mcur = jnp.maximum(mcur, edge_scores(eb, j))
            return mcur

        mvec = lax.fori_loop(0, _NBLK, body_a,
                             jnp.full((16,), -1e30, jnp.float32))
        mbuf_v[...] = mvec
        pltpu.sync_copy(mbuf_v, sh_m.at[s])

        def body_z(r, t):
            for j in range(D // 16):
                rows_v[r, pl.ds(j * 16, 16)] = zero16
            return t

        lax.fori_loop(0, _BE, body_z, 0)
        for j in range(_BE // 16):
            zrow_v[pl.ds(j * 16, 16)] = zero16
        for k in range(_NZC):
            base = s * _ROWS_PT + k * _BE
            pltpu.sync_copy(rows_v, sh_p.at[pl.ds(base, _BE)])
            pltpu.sync_copy(zrow_v, sh_d.at[pl.ds(base, _BE)])

        plsc.subcore_barrier()

        pltpu.sync_copy(sh_m, mall_v)
        macc = mall_v[0, :]
        for t in range(1, 16):
            macc = jnp.maximum(macc, mall_v[t, :])
        M = jnp.max(macc)

        def body_b(b, t):
            eb = ebase + b * _BE
            pltpu.sync_copy(src_h.at[pl.ds(eb, _BE)], srcblk_v)
            pltpu.sync_copy(dst_h.at[pl.ds(eb, _BE)], dstblk_v)
            for j in range(_BE // 16):
                eeblk_v[pl.ds(j * 16, 16)] = jnp.exp(edge_scores(eb, j) - M)
            pltpu.async_copy(feat_h.at[srcblk_v], rows_v, sem).wait()

            def body_r(r, u):
                av = plsc.load_gather(eeblk_v, [jnp.full((16,), r, jnp.int32)])
                for j in range(D // 16):
                    sl = pl.ds(j * 16, 16)
                    rows_v[r, sl] = rows_v[r, sl] * av
                return u

            lax.fori_loop(0, _BE, body_r, 0)
            pltpu.sync_copy(eeblk_v, sh_d.at[dstblk_v], add=True)
            pltpu.sync_copy(rows_v, sh_p.at[dstblk_v], add=True)
            return t

        lax.fori_loop(0, _NBLK, body_b, 0)

        plsc.subcore_barrier()

        for k in range(_NZC):
            base = s * _ROWS_PT + k * _BE
            pltpu.sync_copy(sh_p.at[pl.ds(base, _BE)],
                            p_h.at[c].at[pl.ds(base, _BE)])

        @pl.when(s == 0)
        def _():
            pltpu.sync_copy(sh_d, d_h.at[c])
            mbuf_v[...] = jnp.broadcast_to(M, (16,))
            pltpu.sync_copy(mbuf_v, m_h.at[c])

    return sc_fn


def _tc_pre_body(x_ref, w_ref, alr_ref, feat_ref, eler_ref):
    feat = jnp.dot(x_ref[...], w_ref[...], preferred_element_type=jnp.float32)
    feat_ref[...] = feat
    eler_ref[...] = jnp.dot(feat, alr_ref[...],
                            preferred_element_type=jnp.float32)


_tc_pre = pl.pallas_call(
    _tc_pre_body,
    out_shape=[
        jax.ShapeDtypeStruct((_N, _H), jnp.float32),
        jax.ShapeDtypeStruct((_N, 8), jnp.float32),
    ],
)


def _make_tc_mid(has_res, Dn):
    def body(*refs):
        if has_res:
            (p0, p1, d0, d1, w, hprev, b, wn, alrn,
             h_ref, feat_ref, eler_ref) = refs
        else:
            (p0, p1, d0, d1, w, b, wn, alrn,
             h_ref, feat_ref, eler_ref) = refs
        w0 = w[0]
        w1 = w[1]
        num = w0 * p0[...] + w1 * p1[...]
        den = w0 * d0[...] + w1 * d1[...]
        agg = jnp.where(den > 0.0, num / den, 0.0)
        h = agg + b[...]
        if has_res:
            h = h + hprev[...]
        h = jnp.where(h > 0.0, h, jnp.exp(h) - 1.0)
        h_ref[...] = h
        feat = jnp.dot(h, wn[...], preferred_element_type=jnp.float32)
        feat_ref[...] = feat
        eler_ref[...] = jnp.dot(feat, alrn[...],
                                preferred_element_type=jnp.float32)

    n_in = 9 if has_res else 8
    specs = [pl.BlockSpec(memory_space=pltpu.VMEM) for _ in range(n_in)]
    specs[4] = pl.BlockSpec(memory_space=pltpu.SMEM)
    return pl.pallas_call(
        body,
        in_specs=specs,
        out_shape=[
            jax.ShapeDtypeStruct((_N, _H), jnp.float32),
            jax.ShapeDtypeStruct((_N, Dn), jnp.float32),
            jax.ShapeDtypeStruct((_N, 8), jnp.float32),
        ],
    )


_tc_mid0 = _make_tc_mid(False, _H)
_tc_mid1 = _make_tc_mid(True, _H)


def _tc_fin_body(p0, p1, d0, d1, w, hprev, wres, b, out_ref):
    w0 = w[0]
    w1 = w[1]
    num = w0 * p0[...] + w1 * p1[...]
    den = w0 * d0[...] + w1 * d1[...]
    agg = jnp.where(den > 0.0, num / den, 0.0)
    res = jnp.dot(hprev[...], wres[...], preferred_element_type=jnp.float32)
    out_ref[...] = agg + res + b[...]


_tc_fin_specs = [pl.BlockSpec(memory_space=pltpu.VMEM) for _ in range(8)]
_tc_fin_specs[4] = pl.BlockSpec(memory_space=pltpu.SMEM)
_tc_fin = pl.pallas_call(
    _tc_fin_body,
    in_specs=_tc_fin_specs,
    out_shape=jax.ShapeDtypeStruct((_N, _C), jnp.float32),
)


def _alr(al, ar):
    z = jnp.zeros_like(al)
    return jnp.stack([al, ar, z, z, z, z, z, z], axis=1)


def kernel(inputs, edge_index, W0, al0, ar0, b0, W1, al1, ar1, b1,
           W2, al2, ar2, b2, Wres2):
    src = edge_index[0]
    dst = edge_index[1]
    pad = jnp.zeros((_EP - _E,), jnp.int32)
    srcp = jnp.concatenate([src, pad])
    dstp = jnp.concatenate([dst, pad])

    def combine_w(m):
        mc = m[:, 0]
        return jnp.exp(mc - jnp.max(mc))

    _sc_layer_h = _make_sc_layer(_H)
    zpad = jnp.zeros((_H, _H - _C), jnp.float32)
    W2p = jnp.concatenate([W2, zpad], axis=1)
    al2p = jnp.concatenate([al2, jnp.zeros((_H - _C,), jnp.float32)])
    ar2p = jnp.concatenate([ar2, jnp.zeros((_H - _C,), jnp.float32)])

    feat0, eler0 = _tc_pre(inputs, W0, _alr(al0, ar0))
    p, d, m = _sc_layer_h(feat0, eler0[:, 0], eler0[:, 1], srcp, dstp)
    w = combine_w(m)
    h1, feat1, eler1 = _tc_mid0(p[0, :_N], p[1, :_N], d[0, :_N, None],
                                d[1, :_N, None], w, b0, W1, _alr(al1, ar1))

    p, d, m = _sc_layer_h(feat1, eler1[:, 0], eler1[:, 1], srcp, dstp)
    w = combine_w(m)
    h2, feat2, eler2 = _tc_mid1(p[0, :_N], p[1, :_N], d[0, :_N, None],
                                d[1, :_N, None], w, h1, b1, W2p,
                                _alr(al2p, ar2p))

    p, d, m = _sc_layer_h(feat2, eler2[:, 0], eler2[:, 1], srcp, dstp)
    w = combine_w(m)
    logits = _tc_fin(p[0, :_N, :_C], p[1, :_N, :_C], d[0, :_N, None],
                     d[1, :_N, None], w, h2, Wres2, b2)
    return logits


# speedup vs baseline: 17.1191x; 1.0839x over previous
import functools

import jax
import jax.numpy as jnp
from jax import lax
from jax.experimental import pallas as pl
from jax.experimental.pallas import tpu as pltpu
from jax.experimental.pallas import tpu_sc as plsc

_N = 10000
_E = 320000
_H = 128
_C = 64
_NP = 10240
_EP = 323584
_EPT = _EP // 32
_BE = 128
_NBLK = _EPT // _BE
_ROWS_PT = _NP // 16
_NZC = _ROWS_PT // _BE
_NEG = 0.2


@functools.cache
def _make_sc_layer(D):
    mesh = plsc.VectorSubcoreMesh(core_axis_name="c", subcore_axis_name="s")

    @functools.partial(
        pl.kernel,
        out_type=[
            jax.ShapeDtypeStruct((2, _NP, D), jnp.float32),
            jax.ShapeDtypeStruct((2, _NP), jnp.float32),
        ],
        mesh=mesh,
        scratch_types=[
            pltpu.VMEM((_N,), jnp.float32),
            pltpu.VMEM((_N,), jnp.float32),
            pltpu.VMEM((_BE, D), jnp.float32),
            pltpu.VMEM((_BE,), jnp.int32),
            pltpu.VMEM((_BE,), jnp.int32),
            pltpu.VMEM((_BE,), jnp.float32),
            pltpu.VMEM((_BE,), jnp.float32),
            pltpu.VMEM((16,), jnp.float32),
            pltpu.VMEM_SHARED((_NP, D), jnp.float32),
            pltpu.VMEM_SHARED((_NP,), jnp.float32),
            pltpu.SemaphoreType.DMA,
        ],
        compiler_params=pltpu.CompilerParams(needs_layout_passes=False),
    )
    def sc_fn(feat_h, el_h, er_h, src_h, dst_h, mv_h, p_h, d_h,
              el_v, er_v, rows_v, srcblk_v, dstblk_v, eeblk_v,
              zrow_v, mvec_v, sh_p, sh_d, sem):
        c = lax.axis_index("c")
        s = lax.axis_index("s")
        wid = c * 16 + s
        ebase = wid * _EPT

        pltpu.sync_copy(el_h, el_v)
        pltpu.sync_copy(er_h, er_v)
        pltpu.sync_copy(mv_h, mvec_v)
        Mv = mvec_v[...]

        iota16 = lax.broadcasted_iota(jnp.int32, (16,), 0)
        zero16 = jnp.zeros((16,), jnp.float32)

        def edge_scores(eb, j):
            sl = pl.ds(j * 16, 16)
            e = (plsc.load_gather(el_v, [srcblk_v[sl]])
                 + plsc.load_gather(er_v, [dstblk_v[sl]]))
            e = jnp.where(e >= 0.0, e, _NEG * e)
            gid = eb + j * 16 + iota16
            return jnp.where(gid < _E, e, -1e30)

        def body_z(r, t):
            for j in range(D // 16):
                rows_v[r, pl.ds(j * 16, 16)] = zero16
            return t

        lax.fori_loop(0, _BE, body_z, 0)
        for j in range(_BE // 16):
            zrow_v[pl.ds(j * 16, 16)] = zero16
        for k in range(_NZC):
            base = s * _ROWS_PT + k * _BE
            pltpu.sync_copy(rows_v, sh_p.at[pl.ds(base, _BE)])
            pltpu.sync_copy(zrow_v, sh_d.at[pl.ds(base, _BE)])

        plsc.subcore_barrier()

        def body_b(b, t):
            eb = ebase + b * _BE
            pltpu.sync_copy(src_h.at[pl.ds(eb, _BE)], srcblk_v)
            pltpu.sync_copy(dst_h.at[pl.ds(eb, _BE)], dstblk_v)
            for j in range(_BE // 16):
                eeblk_v[pl.ds(j * 16, 16)] = jnp.exp(edge_scores(eb, j) - Mv)
            pltpu.async_copy(feat_h.at[srcblk_v], rows_v, sem).wait()

            def body_r(r, u):
                av = plsc.load_gather(eeblk_v, [jnp.full((16,), r, jnp.int32)])
                for j in range(D // 16):
                    sl = pl.ds(j * 16, 16)
                    rows_v[r, sl] = rows_v[r, sl] * av
                return u

            lax.fori_loop(0, _BE, body_r, 0)
            pltpu.sync_copy(eeblk_v, sh_d.at[dstblk_v], add=True)
            pltpu.sync_copy(rows_v, sh_p.at[dstblk_v], add=True)
            return t

        lax.fori_loop(0, _NBLK, body_b, 0)

        plsc.subcore_barrier()

        for k in range(_NZC):
            base = s * _ROWS_PT + k * _BE
            pltpu.sync_copy(sh_p.at[pl.ds(base, _BE)],
                            p_h.at[c].at[pl.ds(base, _BE)])

        @pl.when(s == 0)
        def _():
            pltpu.sync_copy(sh_d, d_h.at[c])

    return sc_fn


def _tc_pre_body(x_ref, w_ref, alr_ref, feat_ref, eler_ref, emax_ref):
    feat = jnp.dot(x_ref[...], w_ref[...], preferred_element_type=jnp.float32)
    feat_ref[...] = feat
    eler = jnp.dot(feat, alr_ref[...], preferred_element_type=jnp.float32)
    eler_ref[...] = eler
    emax_ref[...] = jnp.max(eler, axis=0, keepdims=True)


_tc_pre = pl.pallas_call(
    _tc_pre_body,
    out_shape=[
        jax.ShapeDtypeStruct((_N, _H), jnp.float32),
        jax.ShapeDtypeStruct((_N, 8), jnp.float32),
        jax.ShapeDtypeStruct((1, 8), jnp.float32),
    ],
)


def _make_tc_mid(has_res, Dn):
    def body(*refs):
        if has_res:
            (p0, p1, d0, d1, hprev, b, wn, alrn,
             h_ref, feat_ref, eler_ref, emax_ref) = refs
        else:
            (p0, p1, d0, d1, b, wn, alrn,
             h_ref, feat_ref, eler_ref, emax_ref) = refs
        num = p0[...] + p1[...]
        den = d0[...] + d1[...]
        agg = jnp.where(den > 0.0, num / den, 0.0)
        h = agg + b[...]
        if has_res:
            h = h + hprev[...]
        h = jnp.where(h > 0.0, h, jnp.exp(h) - 1.0)
        h_ref[...] = h
        feat = jnp.dot(h, wn[...], preferred_element_type=jnp.float32)
        feat_ref[...] = feat
        eler = jnp.dot(feat, alrn[...], preferred_element_type=jnp.float32)
        eler_ref[...] = eler
        emax_ref[...] = jnp.max(eler, axis=0, keepdims=True)

    return pl.pallas_call(
        body,
        out_shape=[
            jax.ShapeDtypeStruct((_N, _H), jnp.float32),
            jax.ShapeDtypeStruct((_N, Dn), jnp.float32),
            jax.ShapeDtypeStruct((_N, 8), jnp.float32),
            jax.ShapeDtypeStruct((1, 8), jnp.float32),
        ],
    )


_tc_mid0 = _make_tc_mid(False, _H)
_tc_mid1 = _make_tc_mid(True, _H)


def _tc_fin_body(p0, p1, d0, d1, hprev, wres, b, out_ref):
    num = p0[...] + p1[...]
    den = d0[...] + d1[...]
    agg = jnp.where(den > 0.0, num / den, 0.0)
    res = jnp.dot(hprev[...], wres[...], preferred_element_type=jnp.float32)
    out_ref[...] = agg + res + b[...]


_tc_fin = pl.pallas_call(
    _tc_fin_body,
    out_shape=jax.ShapeDtypeStruct((_N, _C), jnp.float32),
)


def _alr(al, ar):
    z = jnp.zeros_like(al)
    return jnp.stack([al, ar, z, z, z, z, z, z], axis=1)


def kernel(inputs, edge_index, W0, al0, ar0, b0, W1, al1, ar1, b1,
           W2, al2, ar2, b2, Wres2):
    src = edge_index[0]
    dst = edge_index[1]
    pad = jnp.zeros((_EP - _E,), jnp.int32)
    srcp = jnp.concatenate([src, pad])
    dstp = jnp.concatenate([dst, pad])

    def shift_vec(emax):
        m0 = emax[0, 0] + emax[0, 1]
        m = jnp.where(m0 >= 0.0, m0, _NEG * m0)
        return jnp.full((16,), m, jnp.float32)

    _sc_layer_h = _make_sc_layer(_H)
    zpad = jnp.zeros((_H, _H - _C), jnp.float32)
    W2p = jnp.concatenate([W2, zpad], axis=1)
    al2p = jnp.concatenate([al2, jnp.zeros((_H - _C,), jnp.float32)])
    ar2p = jnp.concatenate([ar2, jnp.zeros((_H - _C,), jnp.float32)])

    feat0, eler0, emax0 = _tc_pre(inputs, W0, _alr(al0, ar0))
    p, d = _sc_layer_h(feat0, eler0[:, 0], eler0[:, 1], srcp, dstp,
                       shift_vec(emax0))
    h1, feat1, eler1, emax1 = _tc_mid0(p[0, :_N], p[1, :_N], d[0, :_N, None],
                                       d[1, :_N, None], b0, W1,
                                       _alr(al1, ar1))

    p, d = _sc_layer_h(feat1, eler1[:, 0], eler1[:, 1], srcp, dstp,
                       shift_vec(emax1))
    h2, feat2, eler2, emax2 = _tc_mid1(p[0, :_N], p[1, :_N], d[0, :_N, None],
                                       d[1, :_N, None], h1, b1, W2p,
                                       _alr(al2p, ar2p))

    p, d = _sc_layer_h(feat2, eler2[:, 0], eler2[:, 1], srcp, dstp,
                       shift_vec(emax2))
    logits = _tc_fin(p[0, :_N, :_C], p[1, :_N, :_C], d[0, :_N, None],
                     d[1, :_N, None], h2, Wres2, b2)
    return logits
